# baseline (speedup 1.0000x reference)
import functools

import jax
import jax.numpy as jnp
from jax import lax
from jax.experimental import pallas as pl
from jax.experimental.pallas import tpu as pltpu
from jax.experimental.pallas import tpu_sc as plsc

H = 512
W = 512
N = H * W
D = 20
K = 17 * 17 * 17 * 16
KC = 17 * 17 * 17
KP = 5120
RV = 16 * D
NC = 2
NS = 16
NW = NC * NS
PPW = N // NW
L = 16
CB = KP // NW
SUBR = 80
NSUB = CB // SUBR
PART = 2 * D * L

_mesh = plsc.VectorSubcoreMesh(core_axis_name="c", subcore_axis_name="s")


@functools.partial(
    pl.kernel,
    mesh=_mesh,
    out_type=jax.ShapeDtypeStruct((2 * NW * KP,), jnp.float32),
    compiler_params=pltpu.CompilerParams(
        needs_layout_passes=False, use_tc_tiling_on_sc=False),
    scratch_types=[
        pltpu.VMEM((PPW,), jnp.float32),
        pltpu.VMEM((PPW,), jnp.float32),
        pltpu.VMEM((PPW,), jnp.float32),
        pltpu.VMEM((PPW,), jnp.float32),
        pltpu.VMEM((PPW,), jnp.float32),
        pltpu.VMEM((PPW,), jnp.float32),
        pltpu.VMEM((KP,), jnp.float32),
        pltpu.VMEM((KP,), jnp.float32),
        pltpu.VMEM((L,), jnp.float32),
    ],
)
def _hist_kernel(xin_hbm, xs_hbm, w_hbm, out_hbm,
                 a0, a1, a2, b0, b1, b2, hm, hl, wv):
    wid = lax.axis_index("s") * NC + lax.axis_index("c")
    base = wid * PPW

    pltpu.sync_copy(w_hbm, wv)
    pltpu.sync_copy(xin_hbm.at[pl.ds(base, PPW)], a0)
    pltpu.sync_copy(xin_hbm.at[pl.ds(N + base, PPW)], a1)
    pltpu.sync_copy(xin_hbm.at[pl.ds(2 * N + base, PPW)], a2)
    pltpu.sync_copy(xs_hbm.at[pl.ds(base, PPW)], b0)
    pltpu.sync_copy(xs_hbm.at[pl.ds(N + base, PPW)], b1)
    pltpu.sync_copy(xs_hbm.at[pl.ds(2 * N + base, PPW)], b2)

    def zero_body(i, carry):
        z = jnp.zeros((L,), jnp.float32)
        hm[pl.ds(i * L, L)] = z
        hl[pl.ds(i * L, L)] = z
        return carry

    lax.fori_loop(0, KP // L, zero_body, 0)

    wvec = wv[pl.ds(0, L)] * 0.0625
    w0 = wvec[0]
    w1 = wvec[1]
    w2 = wvec[2]
    ones = jnp.ones((L,), jnp.float32)

    def body(i, carry):
        o = i * L
        im = (a0[pl.ds(o, L)] * w0 + a1[pl.ds(o, L)] * w1
              + a2[pl.ds(o, L)] * w2).astype(jnp.int32)
        il = (b0[pl.ds(o, L)] * w0 + b1[pl.ds(o, L)] * w1
              + b2[pl.ds(o, L)] * w2).astype(jnp.int32)
        plsc.addupdate_scatter(hm, [im], ones)
        plsc.addupdate_scatter(hl, [il], ones)
        return carry

    lax.fori_loop(0, PPW // L, body, 0)

    pltpu.sync_copy(hm, out_hbm.at[pl.ds(wid * KP, KP)])
    pltpu.sync_copy(hl, out_hbm.at[pl.ds((NW + wid) * KP, KP)])


@functools.partial(
    pl.kernel,
    mesh=_mesh,
    out_type=jax.ShapeDtypeStruct((NW * PART,), jnp.float32),
    compiler_params=pltpu.CompilerParams(
        needs_layout_passes=False, use_tc_tiling_on_sc=False),
    scratch_types=[
        pltpu.MemorySpace.VMEM_SHARED((2 * NW * KP,), jnp.float32),
        pltpu.VMEM((CB,), jnp.float32),
        pltpu.VMEM((CB,), jnp.float32),
        pltpu.VMEM((CB,), jnp.float32),
        pltpu.VMEM((SUBR * RV,), jnp.float32),
        pltpu.VMEM((PART,), jnp.float32),
    ],
)
def _dot_kernel(hist_hbm, tmf, tlf, parts_hbm,
                shr, tmp, combm, combl, buf, acc):
    c = lax.axis_index("c")
    s = lax.axis_index("s")
    wid = s * NC + c
    mirror = s * NC + (1 - c)

    for w in (wid, mirror):
        pltpu.sync_copy(hist_hbm.at[pl.ds(w * KP, KP)],
                        shr.at[pl.ds(w * KP, KP)])
        pltpu.sync_copy(hist_hbm.at[pl.ds((NW + w) * KP, KP)],
                        shr.at[pl.ds((NW + w) * KP, KP)])
    plsc.subcore_barrier()

    k0b = wid * CB

    def zcomb(i, carry):
        z = jnp.zeros((L,), jnp.float32)
        combm[pl.ds(i * L, L)] = z
        combl[pl.ds(i * L, L)] = z
        return carry

    lax.fori_loop(0, CB // L, zcomb, 0)

    def zacc(i, carry):
        acc[pl.ds(i * L, L)] = jnp.zeros((L,), jnp.float32)
        return carry

    lax.fori_loop(0, PART // L, zacc, 0)

    for w2 in range(NW):
        pltpu.sync_copy(shr.at[pl.ds(w2 * KP + k0b, CB)], tmp)

        def addm(i, carry):
            o = i * L
            combm[pl.ds(o, L)] = combm[pl.ds(o, L)] + tmp[pl.ds(o, L)]
            return carry

        lax.fori_loop(0, CB // L, addm, 0)
        pltpu.sync_copy(shr.at[pl.ds((NW + w2) * KP + k0b, CB)], tmp)

        def addl(i, carry):
            o = i * L
            combl[pl.ds(o, L)] = combl[pl.ds(o, L)] + tmp[pl.ds(o, L)]
            return carry

        lax.fori_loop(0, CB // L, addl, 0)

    for t in range(2):
        tbl = tmf if t == 0 else tlf
        comb = combm if t == 0 else combl
        accbase = t * D * L
        for sub in range(NSUB):
            row0 = k0b + sub * SUBR

            @pl.when(row0 < KC)
            def _window():
                start = jnp.minimum(row0, KC - SUBR)
                pltpu.sync_copy(tbl.at[pl.ds(start * RV, SUBR * RV)], buf)

                def dbody(d, carry):
                    for j in range(SUBR // L):
                        lane = lax.iota(jnp.int32, L)
                        rows = start + j * L + lane
                        cvec = plsc.load_gather(comb, [rows - k0b])
                        cvec = jnp.where(rows >= row0, cvec, 0.0)
                        tv = plsc.load_gather(
                            buf, [(j * L + lane) * RV + d])
                        o = accbase + d * L
                        acc[pl.ds(o, L)] = acc[pl.ds(o, L)] + cvec * tv
                    return carry

                lax.fori_loop(0, D, dbody, 0)

    pltpu.sync_copy(acc, parts_hbm.at[pl.ds(wid * PART, PART)])


def _final_body(p_ref, o_ref):
    v = p_ref[...]
    s0 = jnp.sum(v, axis=0)
    sd = s0[:D, :] + s0[D:, :]
    sc = jnp.sum(sd, axis=1)
    r = sc.reshape(1, D) * (1.0 / N)
    o_ref[...] = jnp.clip(jnp.round(r * 4.0) * 0.25, -32.0, 31.75)


@jax.jit
def kernel(x_in, x_s, feature_msb, feature_lsb, weights):
    xin = x_in.reshape(3 * N)
    xs = x_s.reshape(3 * N)
    wpad = jnp.pad(weights.reshape(3).astype(jnp.float32), (0, L - 3))

    hists = _hist_kernel(xin, xs, wpad)
    parts = _dot_kernel(hists,
                        feature_msb.reshape(K * D),
                        feature_lsb.reshape(K * D))

    out = pl.pallas_call(
        _final_body,
        out_shape=jax.ShapeDtypeStruct((1, D), jnp.float32),
    )(parts.reshape(NW, 2 * D, L))
    return out.reshape(1, D, 1, 1)


# speedup vs baseline: 1.0576x; 1.0576x over previous
import functools

import jax
import jax.numpy as jnp
from jax import lax
from jax.experimental import pallas as pl
from jax.experimental.pallas import tpu as pltpu
from jax.experimental.pallas import tpu_sc as plsc

H = 512
W = 512
N = H * W
D = 20
K = 17 * 17 * 17 * 16
KC = 17 * 17 * 17
KP = 5120
RV = 16 * D
NC = 2
NS = 16
NW = NC * NS
PPW = N // NW
L = 16
CB = KP // NW
SUBR = 80
NSUB = CB // SUBR
PART = 2 * D * L

_mesh = plsc.VectorSubcoreMesh(core_axis_name="c", subcore_axis_name="s")


@functools.partial(
    pl.kernel,
    mesh=_mesh,
    out_type=jax.ShapeDtypeStruct((2 * NW * KP,), jnp.float32),
    compiler_params=pltpu.CompilerParams(
        needs_layout_passes=False, use_tc_tiling_on_sc=False),
    scratch_types=[
        pltpu.VMEM((PPW,), jnp.float32),
        pltpu.VMEM((PPW,), jnp.float32),
        pltpu.VMEM((PPW,), jnp.float32),
        pltpu.VMEM((PPW,), jnp.float32),
        pltpu.VMEM((PPW,), jnp.float32),
        pltpu.VMEM((PPW,), jnp.float32),
        pltpu.VMEM((KP,), jnp.float32),
        pltpu.VMEM((KP,), jnp.float32),
        pltpu.VMEM((L,), jnp.float32),
    ],
)
def _hist_kernel(xin_hbm, xs_hbm, w_hbm, out_hbm,
                 a0, a1, a2, b0, b1, b2, hm, hl, wv):
    wid = lax.axis_index("s") * NC + lax.axis_index("c")
    base = wid * PPW

    pltpu.sync_copy(w_hbm, wv)
    pltpu.sync_copy(xin_hbm.at[pl.ds(base, PPW)], a0)
    pltpu.sync_copy(xin_hbm.at[pl.ds(N + base, PPW)], a1)
    pltpu.sync_copy(xin_hbm.at[pl.ds(2 * N + base, PPW)], a2)
    pltpu.sync_copy(xs_hbm.at[pl.ds(base, PPW)], b0)
    pltpu.sync_copy(xs_hbm.at[pl.ds(N + base, PPW)], b1)
    pltpu.sync_copy(xs_hbm.at[pl.ds(2 * N + base, PPW)], b2)

    def zero_body(i, carry):
        z = jnp.zeros((L,), jnp.float32)
        hm[pl.ds(i * L, L)] = z
        hl[pl.ds(i * L, L)] = z
        return carry

    lax.fori_loop(0, KP // L, zero_body, 0)

    wvec = wv[pl.ds(0, L)] * 0.0625
    w0 = wvec[0]
    w1 = wvec[1]
    w2 = wvec[2]
    ones = jnp.ones((L,), jnp.float32)

    def body(i, carry):
        o = i * L
        im = (a0[pl.ds(o, L)] * w0 + a1[pl.ds(o, L)] * w1
              + a2[pl.ds(o, L)] * w2).astype(jnp.int32)
        il = (b0[pl.ds(o, L)] * w0 + b1[pl.ds(o, L)] * w1
              + b2[pl.ds(o, L)] * w2).astype(jnp.int32)
        plsc.addupdate_scatter(hm, [im], ones)
        plsc.addupdate_scatter(hl, [il], ones)
        return carry

    lax.fori_loop(0, PPW // L, body, 0)

    pltpu.sync_copy(hm, out_hbm.at[pl.ds(wid * KP, KP)])
    pltpu.sync_copy(hl, out_hbm.at[pl.ds((NW + wid) * KP, KP)])


@functools.partial(
    pl.kernel,
    mesh=_mesh,
    out_type=jax.ShapeDtypeStruct((NW * PART,), jnp.float32),
    compiler_params=pltpu.CompilerParams(
        needs_layout_passes=False, use_tc_tiling_on_sc=False),
    scratch_types=[
        pltpu.MemorySpace.VMEM_SHARED((NW, KP), jnp.float32),
        pltpu.MemorySpace.VMEM_SHARED((NW, KP), jnp.float32),
        pltpu.VMEM((NW, CB), jnp.float32),
        pltpu.VMEM((NW, CB), jnp.float32),
        pltpu.VMEM((CB,), jnp.float32),
        pltpu.VMEM((CB,), jnp.float32),
        pltpu.VMEM((SUBR * RV,), jnp.float32),
        pltpu.VMEM((PART,), jnp.float32),
    ],
)
def _dot_kernel(hist_hbm, tmf, tlf, parts_hbm,
                shrm, shrl, bufm, bufl, combm, combl, buf, acc):
    c = lax.axis_index("c")
    s = lax.axis_index("s")
    wid = s * NC + c
    mirror = s * NC + (1 - c)

    for w in (wid, mirror):
        pltpu.sync_copy(hist_hbm.at[pl.ds(w, 1)], shrm.at[pl.ds(w, 1)])
        pltpu.sync_copy(hist_hbm.at[pl.ds(NW + w, 1)], shrl.at[pl.ds(w, 1)])
    plsc.subcore_barrier()

    k0b = wid * CB

    def zacc(i, carry):
        acc[pl.ds(i * L, L)] = jnp.zeros((L,), jnp.float32)
        return carry

    lax.fori_loop(0, PART // L, zacc, 0)

    pltpu.sync_copy(shrm.at[:, pl.ds(k0b, CB)], bufm)
    pltpu.sync_copy(shrl.at[:, pl.ds(k0b, CB)], bufl)

    def csum(i, carry):
        o = i * L
        vm = bufm[0, pl.ds(o, L)]
        vl = bufl[0, pl.ds(o, L)]
        for w2 in range(1, NW):
            vm = vm + bufm[w2, pl.ds(o, L)]
            vl = vl + bufl[w2, pl.ds(o, L)]
        combm[pl.ds(o, L)] = vm
        combl[pl.ds(o, L)] = vl
        return carry

    lax.fori_loop(0, CB // L, csum, 0)

    for t in range(2):
        tbl = tmf if t == 0 else tlf
        comb = combm if t == 0 else combl
        accbase = t * D * L
        for sub in range(NSUB):
            row0 = k0b + sub * SUBR

            @pl.when(row0 < KC)
            def _window():
                start = jnp.minimum(row0, KC - SUBR)
                pltpu.sync_copy(tbl.at[pl.ds(start * RV, SUBR * RV)], buf)

                def dbody(d, carry):
                    for j in range(SUBR // L):
                        lane = lax.iota(jnp.int32, L)
                        rows = start + j * L + lane
                        cvec = plsc.load_gather(comb, [rows - k0b])
                        cvec = jnp.where(rows >= row0, cvec, 0.0)
                        tv = plsc.load_gather(
                            buf, [(j * L + lane) * RV + d])
                        o = accbase + d * L
                        acc[pl.ds(o, L)] = acc[pl.ds(o, L)] + cvec * tv
                    return carry

                lax.fori_loop(0, D, dbody, 0)

    pltpu.sync_copy(acc, parts_hbm.at[pl.ds(wid * PART, PART)])


def _final_body(p_ref, o_ref):
    v = p_ref[...]
    s0 = jnp.sum(v, axis=0)
    sd = s0[:D, :] + s0[D:, :]
    sc = jnp.sum(sd, axis=1)
    r = sc.reshape(1, D) * (1.0 / N)
    o_ref[...] = jnp.clip(jnp.round(r * 4.0) * 0.25, -32.0, 31.75)


@jax.jit
def kernel(x_in, x_s, feature_msb, feature_lsb, weights):
    xin = x_in.reshape(3 * N)
    xs = x_s.reshape(3 * N)
    wpad = jnp.pad(weights.reshape(3).astype(jnp.float32), (0, L - 3))

    hists = _hist_kernel(xin, xs, wpad)
    parts = _dot_kernel(hists.reshape(2 * NW, KP),
                        feature_msb.reshape(K * D),
                        feature_lsb.reshape(K * D))

    out = pl.pallas_call(
        _final_body,
        out_shape=jax.ShapeDtypeStruct((1, D), jnp.float32),
    )(parts.reshape(NW, 2 * D, L))
    return out.reshape(1, D, 1, 1)
